# Initial kernel scaffold; baseline (speedup 1.0000x reference)
import jax
import jax.numpy as jnp
from jax.experimental import pallas as pl
from jax.experimental.pallas import tpu as pltpu

_N = 2048
_NBIN = 64
_MPB = 256
_CUT = 0.2
_TILE = 256
_NT = _N // _TILE
_NCLS = 12

_SELU_SCALE = 1.0507009873554805
_SELU_ALPHA = 1.6732632423543772


def _selu(x):
    return _SELU_SCALE * jnp.where(x > 0, x, _SELU_ALPHA * jnp.expm1(x))


def _mm(a, b):
    return jnp.dot(a, b, preferred_element_type=jnp.float32)


def _dotT(a, b):
    return jax.lax.dot_general(
        a, b, (((1,), (1,)), ((), ())), preferred_element_type=jnp.float32)


def _prep_body(x_ref, wd_ref, bd_ref, pts_ref, bv_ref):
    x = x_ref[0]
    ids = x[:, 0:1].astype(jnp.int32)
    ioh = jax.lax.broadcasted_iota(jnp.int32, (_N, _NCLS), 1)
    oh = (ids == ioh).astype(jnp.float32)
    enc = jnp.concatenate([oh, x[:, 1:]], axis=1)
    xd = _selu(_mm(enc, wd_ref[...]) + bd_ref[...])
    lsh = xd[:, :_NBIN]
    pts = xd[:, _NBIN:]
    mx = jnp.max(lsh, axis=1, keepdims=True)
    i64 = jax.lax.broadcasted_iota(jnp.int32, (_N, _NBIN), 1)
    cand = jnp.where(lsh == mx, i64, _NBIN)
    b = jnp.min(cand, axis=1, keepdims=True)
    boh = (i64 == b).astype(jnp.float32)
    r_i = jax.lax.broadcasted_iota(jnp.int32, (_TILE, _TILE), 0)
    c_i = jax.lax.broadcasted_iota(jnp.int32, (_TILE, _TILE), 1)
    ltri = (r_i > c_i).astype(jnp.float32)
    offs = jnp.zeros((1, _NBIN), jnp.float32)
    ranks = []
    for c in range(_NT):
        blk = boh[c * _TILE:(c + 1) * _TILE]
        ranks.append(_mm(ltri, blk) + offs)
        offs = offs + jnp.sum(blk, axis=0, keepdims=True)
    rank_mat = jnp.concatenate(ranks, axis=0)
    rank = jnp.sum(rank_mat * boh, axis=1, keepdims=True)
    valid = rank < float(_MPB)
    bv_ref[0] = jnp.where(valid, boh, 0.0)
    pts_ref[0] = pts


def _adj_body(ptsr_ref, bvr_ref, pts_ref, bv_ref, adj_ref, deg_ref):
    rows = ptsr_ref[0]
    bvr = bvr_ref[0]
    ptsf = pts_ref[0]
    bvf = bv_ref[0]
    vm = _dotT(bvr, bvf) > 0.5
    sim = _dotT(rows, ptsf)
    sim = jnp.where(vm, sim, -1e9)
    m = jnp.max(sim, axis=1, keepdims=True)
    e = jnp.exp(sim - m)
    d = e / jnp.sum(e, axis=1, keepdims=True)
    d = jnp.where(vm, d, 0.0)
    a = jnp.where(d > _CUT, jnp.exp(-d), 0.0)
    adj_ref[0] = a
    deg_ref[0] = jnp.sum(a, axis=1, keepdims=True)


def _main_body(x_ref, adj_ref, deg_ref,
               W1, b1, W2, b2, W3, b3,
               c1Wt, c1bt, c1Wh, c1th,
               id1W, id1b, id2W, id2b, id3W, id3b,
               idW, idb, chW, chb,
               c2Wt, c2bt, c2Wh, c2th,
               m1W, m1b, m2W, m2b, m3W, m3b,
               mo1W, mo1b, mo2W, mo2b, mo3W, mo3b, moW, mob,
               out_ref):
    x_in = x_ref[0]
    ids = x_in[:, 0:1].astype(jnp.int32)
    ioh = jax.lax.broadcasted_iota(jnp.int32, (_N, _NCLS), 1)
    oh = (ids == ioh).astype(jnp.float32)
    enc = jnp.concatenate([oh, x_in[:, 1:]], axis=1)
    adj = adj_ref[0]
    norm = jax.lax.rsqrt(deg_ref[0] + 1e-6)
    x = _selu(_mm(enc, W1[...]) + b1[...])
    x = _selu(_mm(x, W2[...]) + b2[...])
    x = _selu(_mm(x, W3[...]) + b3[...])
    fh = _mm(x, c1th[...]) * norm
    agg = _mm(adj, fh) * norm
    gate = jax.nn.sigmoid(_mm(x, c1Wt[...]) + c1bt[...])
    xg = gate * agg + (1.0 - gate) * _mm(x, c1Wh[...])
    a = _selu(_mm(xg, id1W[...]) + id1b[...])
    a = _selu(_mm(a, id2W[...]) + id2b[...])
    a = _selu(_mm(a, id3W[...]) + id3b[...])
    logits = _mm(a, idW[...]) + idb[...]
    charge = _mm(a, chW[...]) + chb[...]
    lm = jnp.max(logits, axis=1, keepdims=True)
    le = jnp.exp(logits - lm)
    sm = le / jnp.sum(le, axis=1, keepdims=True)
    x2 = jnp.concatenate([xg, sm], axis=1)
    fh2 = _mm(x2, c2th[...]) * norm
    agg2 = _mm(adj, fh2) * norm
    gate2 = jax.nn.sigmoid(_mm(x2, c2Wt[...]) + c2bt[...])
    x2g = gate2 * agg2 + (1.0 - gate2) * _mm(x2, c2Wh[...])
    xm = _selu(_mm(enc, m1W[...]) + m1b[...])
    xm = _selu(_mm(xm, m2W[...]) + m2b[...])
    xm = _selu(_mm(xm, m3W[...]) + m3b[...])
    d = jnp.concatenate([x2g, xm], axis=1)
    d = _selu(_mm(d, mo1W[...]) + mo1b[...])
    d = _selu(_mm(d, mo2W[...]) + mo2b[...])
    d = _selu(_mm(d, mo3W[...]) + mo3b[...])
    mom = _mm(d, moW[...]) + mob[...]
    out_ref[0] = jnp.concatenate([logits, charge, mom], axis=1)


def _wspec(arr):
    return pl.BlockSpec(arr.shape, lambda *_: (0,) * arr.ndim)


def kernel(X, params):
    p = params
    B = X.shape[0]
    f32 = jnp.float32

    def rb(v):
        return v.reshape(1, -1)

    pts, bv = pl.pallas_call(
        _prep_body,
        grid=(B,),
        in_specs=[
            pl.BlockSpec((1, _N, 15), lambda b: (b, 0, 0)),
            _wspec(p['Wd']),
            pl.BlockSpec((1, 256), lambda b: (0, 0)),
        ],
        out_specs=[
            pl.BlockSpec((1, _N, 192), lambda b: (b, 0, 0)),
            pl.BlockSpec((1, _N, _NBIN), lambda b: (b, 0, 0)),
        ],
        out_shape=[
            jax.ShapeDtypeStruct((B, _N, 192), f32),
            jax.ShapeDtypeStruct((B, _N, _NBIN), f32),
        ],
    )(X, p['Wd'], rb(p['bd']))

    adj, deg = pl.pallas_call(
        _adj_body,
        grid=(B, _NT),
        in_specs=[
            pl.BlockSpec((1, _TILE, 192), lambda b, t: (b, t, 0)),
            pl.BlockSpec((1, _TILE, _NBIN), lambda b, t: (b, t, 0)),
            pl.BlockSpec((1, _N, 192), lambda b, t: (b, 0, 0)),
            pl.BlockSpec((1, _N, _NBIN), lambda b, t: (b, 0, 0)),
        ],
        out_specs=[
            pl.BlockSpec((1, _TILE, _N), lambda b, t: (b, t, 0)),
            pl.BlockSpec((1, _TILE, 1), lambda b, t: (b, t, 0)),
        ],
        out_shape=[
            jax.ShapeDtypeStruct((B, _N, _N), f32),
            jax.ShapeDtypeStruct((B, _N, 1), f32),
        ],
    )(pts, bv, pts, bv)

    weights = [
        p['W1'], rb(p['b1']), p['W2'], rb(p['b2']), p['W3'], rb(p['b3']),
        p['c1_Wt'], rb(p['c1_bt']), p['c1_Wh'], p['c1_theta'],
        p['id1_W'], rb(p['id1_b']), p['id2_W'], rb(p['id2_b']),
        p['id3_W'], rb(p['id3_b']),
        p['idW'], rb(p['idb']), p['chW'], rb(p['chb']),
        p['c2_Wt'], rb(p['c2_bt']), p['c2_Wh'], p['c2_theta'],
        p['m1_W'], rb(p['m1_b']), p['m2_W'], rb(p['m2_b']),
        p['m3_W'], rb(p['m3_b']),
        p['mo1_W'], rb(p['mo1_b']), p['mo2_W'], rb(p['mo2_b']),
        p['mo3_W'], rb(p['mo3_b']), p['moW'], rb(p['mob']),
    ]
    out = pl.pallas_call(
        _main_body,
        grid=(B,),
        in_specs=[
            pl.BlockSpec((1, _N, 15), lambda b: (b, 0, 0)),
            pl.BlockSpec((1, _N, _N), lambda b: (b, 0, 0)),
            pl.BlockSpec((1, _N, 1), lambda b: (b, 0, 0)),
        ] + [_wspec(w) for w in weights],
        out_specs=pl.BlockSpec((1, _N, 12), lambda b: (b, 0, 0)),
        out_shape=jax.ShapeDtypeStruct((B, _N, 12), f32),
    )(X, adj, deg, *weights)
    return out


# speedup vs baseline: 197.3742x; 197.3742x over previous
import jax
import jax.numpy as jnp
from jax.experimental import pallas as pl
from jax.experimental.pallas import tpu as pltpu

_N = 2048
_NBIN = 64
_MPB = 256
_CUT = 0.2
_TILE = 256
_NT = _N // _TILE
_NCLS = 12
_D1 = 512
_D2 = 520

_SELU_SCALE = 1.0507009873554805
_SELU_ALPHA = 1.6732632423543772


def _selu(x):
    return _SELU_SCALE * jnp.where(x > 0, x, _SELU_ALPHA * (jnp.exp(x) - 1.0))


def _mm(a, b):
    return jnp.dot(a, b, preferred_element_type=jnp.float32)


def _dotT(a, b):
    return jax.lax.dot_general(
        a, b, (((1,), (1,)), ((), ())), preferred_element_type=jnp.float32)


def _encode(x):
    ids = x[:, 0:1].astype(jnp.int32)
    ioh = jax.lax.broadcasted_iota(jnp.int32, (_N, _NCLS), 1)
    oh = (ids == ioh).astype(jnp.float32)
    return jnp.concatenate([oh, x[:, 1:]], axis=1)


def _prep_body(x_ref, wd_ref, bd_ref, pts_ref, bv_ref):
    enc = _encode(x_ref[0])
    xd = _selu(_mm(enc, wd_ref[...]) + bd_ref[...])
    lsh = xd[:, :_NBIN]
    pts = xd[:, _NBIN:]
    mx = jnp.max(lsh, axis=1, keepdims=True)
    i64 = jax.lax.broadcasted_iota(jnp.int32, (_N, _NBIN), 1)
    cand = jnp.where(lsh == mx, i64, _NBIN)
    b = jnp.min(cand, axis=1, keepdims=True)
    boh = (i64 == b).astype(jnp.float32)
    r_i = jax.lax.broadcasted_iota(jnp.int32, (_TILE, _TILE), 0)
    c_i = jax.lax.broadcasted_iota(jnp.int32, (_TILE, _TILE), 1)
    ltri = (r_i > c_i).astype(jnp.float32)
    offs = jnp.zeros((1, _NBIN), jnp.float32)
    ranks = []
    for c in range(_NT):
        blk = boh[c * _TILE:(c + 1) * _TILE]
        ranks.append(_mm(ltri, blk) + offs)
        offs = offs + jnp.sum(blk, axis=0, keepdims=True)
    rank_mat = jnp.concatenate(ranks, axis=0)
    rank = jnp.sum(rank_mat * boh, axis=1, keepdims=True)
    valid = rank < float(_MPB)
    bv_ref[0] = jnp.where(valid, boh, 0.0)
    pts_ref[0] = pts


def _adj_body(ptsr_ref, bvr_ref, pts_ref, bv_ref, adj_ref, deg_ref):
    rows = ptsr_ref[0]
    bvr = bvr_ref[0]
    ptsf = pts_ref[0]
    bvf = bv_ref[0]
    vm = _dotT(bvr, bvf) > 0.5
    sim = _dotT(rows, ptsf)
    sim = jnp.where(vm, sim, -1e9)
    m = jnp.max(sim, axis=1, keepdims=True)
    e = jnp.exp(sim - m)
    d = e / jnp.sum(e, axis=1, keepdims=True)
    d = jnp.where(vm, d, 0.0)
    a = jnp.where(d > _CUT, jnp.exp(-d), 0.0)
    adj_ref[0] = a
    deg_ref[0] = jnp.sum(a, axis=1, keepdims=True)


def _mlp1_body(x_ref, deg_ref, W1, b1, W2, b2, W3, b3,
               c1Wt, c1bt, c1Wh, c1th,
               fhn_ref, gn_ref, base_ref):
    enc = _encode(x_ref[0])
    norm = jax.lax.rsqrt(deg_ref[0] + 1e-6)
    x = _selu(_mm(enc, W1[...]) + b1[...])
    x = _selu(_mm(x, W2[...]) + b2[...])
    x = _selu(_mm(x, W3[...]) + b3[...])
    gate = jax.nn.sigmoid(_mm(x, c1Wt[...]) + c1bt[...])
    fhn_ref[0] = _mm(x, c1th[...]) * norm
    gn_ref[0] = gate * norm
    base_ref[0] = (1.0 - gate) * _mm(x, c1Wh[...])


def _gh_body(adj_ref, fhn_ref, gn_ref, base_ref, out_ref):
    agg = _mm(adj_ref[0], fhn_ref[0])
    out_ref[0] = gn_ref[0] * agg + base_ref[0]


def _head_body(xg_ref, deg_ref,
               id1W, id1b, id2W, id2b, id3W, id3b,
               idW, idb, chW, chb,
               c2Wt, c2bt, c2Wh, c2th,
               lc_ref, fhn_ref, gn_ref, base_ref):
    xg = xg_ref[0]
    norm = jax.lax.rsqrt(deg_ref[0] + 1e-6)
    a = _selu(_mm(xg, id1W[...]) + id1b[...])
    a = _selu(_mm(a, id2W[...]) + id2b[...])
    a = _selu(_mm(a, id3W[...]) + id3b[...])
    logits = _mm(a, idW[...]) + idb[...]
    charge = _mm(a, chW[...]) + chb[...]
    lm = jnp.max(logits, axis=1, keepdims=True)
    le = jnp.exp(logits - lm)
    sm = le / jnp.sum(le, axis=1, keepdims=True)
    x2 = jnp.concatenate([xg, sm], axis=1)
    gate = jax.nn.sigmoid(_mm(x2, c2Wt[...]) + c2bt[...])
    lc_ref[0] = jnp.concatenate([logits, charge], axis=1)
    fhn_ref[0] = _mm(x2, c2th[...]) * norm
    gn_ref[0] = gate * norm
    base_ref[0] = (1.0 - gate) * _mm(x2, c2Wh[...])


def _tail_body(x_ref, x2g_ref, lc_ref,
               m1W, m1b, m2W, m2b, m3W, m3b,
               mo1W, mo1b, mo2W, mo2b, mo3W, mo3b, moW, mob,
               out_ref):
    enc = _encode(x_ref[0])
    xm = _selu(_mm(enc, m1W[...]) + m1b[...])
    xm = _selu(_mm(xm, m2W[...]) + m2b[...])
    xm = _selu(_mm(xm, m3W[...]) + m3b[...])
    d = jnp.concatenate([x2g_ref[0], xm], axis=1)
    d = _selu(_mm(d, mo1W[...]) + mo1b[...])
    d = _selu(_mm(d, mo2W[...]) + mo2b[...])
    d = _selu(_mm(d, mo3W[...]) + mo3b[...])
    mom = _mm(d, moW[...]) + mob[...]
    out_ref[0] = jnp.concatenate([lc_ref[0], mom], axis=1)


def _wspec(arr):
    return pl.BlockSpec(arr.shape, lambda *_: (0,) * arr.ndim)


def _full(d):
    return pl.BlockSpec((1, _N, d), lambda b: (b, 0, 0))


def kernel(X, params):
    p = params
    B = X.shape[0]
    f32 = jnp.float32

    def rb(v):
        return v.reshape(1, -1)

    def shp(d):
        return jax.ShapeDtypeStruct((B, _N, d), f32)

    pts, bv = pl.pallas_call(
        _prep_body,
        grid=(B,),
        in_specs=[_full(15), _wspec(p['Wd']), pl.BlockSpec((1, 256), lambda b: (0, 0))],
        out_specs=[_full(192), _full(_NBIN)],
        out_shape=[shp(192), shp(_NBIN)],
    )(X, p['Wd'], rb(p['bd']))

    adj, deg = pl.pallas_call(
        _adj_body,
        grid=(B, _NT),
        in_specs=[
            pl.BlockSpec((1, _TILE, 192), lambda b, t: (b, t, 0)),
            pl.BlockSpec((1, _TILE, _NBIN), lambda b, t: (b, t, 0)),
            pl.BlockSpec((1, _N, 192), lambda b, t: (b, 0, 0)),
            pl.BlockSpec((1, _N, _NBIN), lambda b, t: (b, 0, 0)),
        ],
        out_specs=[
            pl.BlockSpec((1, _TILE, _N), lambda b, t: (b, t, 0)),
            pl.BlockSpec((1, _TILE, 1), lambda b, t: (b, t, 0)),
        ],
        out_shape=[shp(_N), shp(1)],
    )(pts, bv, pts, bv)

    w1 = [p['W1'], rb(p['b1']), p['W2'], rb(p['b2']), p['W3'], rb(p['b3']),
          p['c1_Wt'], rb(p['c1_bt']), p['c1_Wh'], p['c1_theta']]
    fhn1, gn1, base1 = pl.pallas_call(
        _mlp1_body,
        grid=(B,),
        in_specs=[_full(15), _full(1)] + [_wspec(w) for w in w1],
        out_specs=[_full(_D1), _full(_D1), _full(_D1)],
        out_shape=[shp(_D1), shp(_D1), shp(_D1)],
    )(X, deg, *w1)

    def gh(adj, fhn, gn, base, d):
        return pl.pallas_call(
            _gh_body,
            grid=(B, _NT),
            in_specs=[
                pl.BlockSpec((1, _TILE, _N), lambda b, t: (b, t, 0)),
                pl.BlockSpec((1, _N, d), lambda b, t: (b, 0, 0)),
                pl.BlockSpec((1, _TILE, d), lambda b, t: (b, t, 0)),
                pl.BlockSpec((1, _TILE, d), lambda b, t: (b, t, 0)),
            ],
            out_specs=pl.BlockSpec((1, _TILE, d), lambda b, t: (b, t, 0)),
            out_shape=shp(d),
        )(adj, fhn, gn, base)

    xg = gh(adj, fhn1, gn1, base1, _D1)

    w2 = [p['id1_W'], rb(p['id1_b']), p['id2_W'], rb(p['id2_b']),
          p['id3_W'], rb(p['id3_b']), p['idW'], rb(p['idb']),
          p['chW'], rb(p['chb']),
          p['c2_Wt'], rb(p['c2_bt']), p['c2_Wh'], p['c2_theta']]
    lc, fhn2, gn2, base2 = pl.pallas_call(
        _head_body,
        grid=(B,),
        in_specs=[_full(_D1), _full(1)] + [_wspec(w) for w in w2],
        out_specs=[_full(9), _full(_D2), _full(_D2), _full(_D2)],
        out_shape=[shp(9), shp(_D2), shp(_D2), shp(_D2)],
    )(xg, deg, *w2)

    x2g = gh(adj, fhn2, gn2, base2, _D2)

    w3 = [p['m1_W'], rb(p['m1_b']), p['m2_W'], rb(p['m2_b']),
          p['m3_W'], rb(p['m3_b']),
          p['mo1_W'], rb(p['mo1_b']), p['mo2_W'], rb(p['mo2_b']),
          p['mo3_W'], rb(p['mo3_b']), p['moW'], rb(p['mob'])]
    out = pl.pallas_call(
        _tail_body,
        grid=(B,),
        in_specs=[_full(15), _full(_D2), _full(9)] + [_wspec(w) for w in w3],
        out_specs=_full(12),
        out_shape=shp(12),
    )(X, x2g, lc, *w3)
    return out
